# Initial kernel scaffold; baseline (speedup 1.0000x reference)
import functools

import jax
import jax.numpy as jnp
from jax import lax
from jax.experimental import pallas as pl
from jax.experimental.pallas import tpu as pltpu
from jax.experimental.pallas import tpu_sc as plsc

B = 4096
L = 200
D = 64
VOCAB = 100000

NUM_CORES = 2
NUM_SUBCORES = 16
NW = NUM_CORES * NUM_SUBCORES
B_PER_W = B // NW
HALF = L // 2


def _emb_body(idx_hbm, table_hbm, pos_hbm, out_hbm, pos_v, idx_v, rows_v, sem):
    cid = lax.axis_index("c")
    sid = lax.axis_index("s")
    wid = sid * NUM_CORES + cid
    b0 = wid * B_PER_W

    pltpu.sync_copy(pos_hbm, pos_v)

    @pl.loop(0, B_PER_W)
    def _chunk(i):
        b = b0 + i
        pltpu.sync_copy(idx_hbm.at[pl.ds(2 * b, 2)], idx_v)
        g0 = pltpu.async_copy(
            table_hbm.at[idx_v.at[0]], rows_v.at[pl.ds(0, HALF)], sem)
        g1 = pltpu.async_copy(
            table_hbm.at[idx_v.at[1]], rows_v.at[pl.ds(HALF, HALF)], sem)
        g0.wait()
        g1.wait()

        @pl.loop(0, L)
        def _add(r):
            for k in range(D // 16):
                sl = pl.ds(k * 16, 16)
                plsc.addupdate(rows_v.at[r, sl], pos_v[r, sl])

        pltpu.sync_copy(rows_v, out_hbm.at[b])


def _emb(idx2, token_table, pos_table):
    mesh = plsc.VectorSubcoreMesh(
        core_axis_name="c", subcore_axis_name="s",
        num_cores=NUM_CORES, num_subcores=NUM_SUBCORES)
    return pl.kernel(
        _emb_body,
        out_type=jax.ShapeDtypeStruct((B, L, D), jnp.float32),
        mesh=mesh,
        scratch_types=[
            pltpu.VMEM((L, D), jnp.float32),
            pltpu.VMEM((2, HALF), jnp.int32),
            pltpu.VMEM((L, D), jnp.float32),
            pltpu.SemaphoreType.DMA,
        ],
    )(idx2, token_table, pos_table)


def kernel(input, token_table, pos_table):
    idx2 = input.reshape(B * L // HALF, HALF)
    return _emb(idx2, token_table, pos_table)


# speedup vs baseline: 3.0985x; 3.0985x over previous
import functools

import jax
import jax.numpy as jnp
from jax import lax
from jax.experimental import pallas as pl
from jax.experimental.pallas import tpu as pltpu
from jax.experimental.pallas import tpu_sc as plsc

B = 4096
L = 200
D = 64
VOCAB = 100000

NUM_CORES = 2
NUM_SUBCORES = 16
NW = NUM_CORES * NUM_SUBCORES
B_PER_W = B // NW
HALF = L // 2


def _emb_body(idx_hbm, table_hbm, pos_hbm, out_hbm, pos_v, idx_v, rows_v, sem):
    cid = lax.axis_index("c")
    sid = lax.axis_index("s")
    wid = sid * NUM_CORES + cid
    b0 = wid * B_PER_W

    pltpu.sync_copy(pos_hbm, pos_v)

    @pl.loop(0, B_PER_W)
    def _chunk(i):
        b = b0 + i
        pltpu.sync_copy(idx_hbm.at[pl.ds(2 * b, 2)], idx_v)
        g0 = pltpu.async_copy(
            table_hbm.at[idx_v.at[0]], rows_v.at[pl.ds(0, HALF)], sem)
        g1 = pltpu.async_copy(
            table_hbm.at[idx_v.at[1]], rows_v.at[pl.ds(HALF, HALF)], sem)
        g0.wait()
        g1.wait()

        @pl.loop(0, L)
        def _add(r):
            for k in range(D // 16):
                sl = pl.ds(k * 16, 16)
                plsc.addupdate(rows_v.at[r, sl], pos_v[r, sl])

        pltpu.sync_copy(rows_v, out_hbm.at[b])


def _emb(idx2, token_table, pos_table):
    mesh = plsc.VectorSubcoreMesh(
        core_axis_name="c", subcore_axis_name="s",
        num_cores=NUM_CORES, num_subcores=NUM_SUBCORES)
    return pl.kernel(
        _emb_body,
        out_type=jax.ShapeDtypeStruct((B, L, D), jnp.float32),
        mesh=mesh,
        scratch_types=[
            pltpu.VMEM((L, D), jnp.float32),
            pltpu.VMEM((2, HALF), jnp.int32),
            pltpu.VMEM((L, D), jnp.float32),
            pltpu.SemaphoreType.DMA,
        ],
        compiler_params=pltpu.CompilerParams(use_tc_tiling_on_sc=False),
    )(idx2, token_table, pos_table)


def kernel(input, token_table, pos_table):
    idx2 = input.reshape(B * L // HALF, HALF)
    return _emb(idx2, token_table, pos_table)


# speedup vs baseline: 4.2051x; 1.3571x over previous
import jax
import jax.numpy as jnp
from jax import lax
from jax.experimental import pallas as pl
from jax.experimental.pallas import tpu as pltpu
from jax.experimental.pallas import tpu_sc as plsc

B = 4096
L = 200
D = 64

NUM_CORES = 2
NUM_SUBCORES = 16
NW = NUM_CORES * NUM_SUBCORES
B_PER_W = B // NW
HALF = L // 2
NBUF = 4


def _emb_body(idx_hbm, table_hbm, pos_hbm, out_hbm,
              pos_v, idx_v, rows_v, gsem, osem):
    cid = lax.axis_index("c")
    sid = lax.axis_index("s")
    wid = sid * NUM_CORES + cid
    b0 = wid * B_PER_W

    pltpu.sync_copy(pos_hbm, pos_v)
    pltpu.sync_copy(idx_hbm.at[pl.ds(2 * b0, 2 * B_PER_W)], idx_v)

    def start_gather(i, j):
        pltpu.async_copy(table_hbm.at[idx_v.at[2 * i]],
                         rows_v.at[j, pl.ds(0, HALF)], gsem.at[j])
        pltpu.async_copy(table_hbm.at[idx_v.at[2 * i + 1]],
                         rows_v.at[j, pl.ds(HALF, HALF)], gsem.at[j])

    def wait_gather(j):
        pltpu.make_async_copy(table_hbm.at[idx_v.at[0]],
                              rows_v.at[j, pl.ds(0, HALF)], gsem.at[j]).wait()
        pltpu.make_async_copy(table_hbm.at[idx_v.at[0]],
                              rows_v.at[j, pl.ds(HALF, HALF)], gsem.at[j]).wait()

    def start_out(i, j):
        pltpu.async_copy(rows_v.at[j], out_hbm.at[b0 + i], osem.at[j])

    def wait_out(j):
        pltpu.make_async_copy(rows_v.at[j], out_hbm.at[b0], osem.at[j]).wait()

    def add_pos(j):
        @pl.loop(0, L, unroll=8)
        def _add(r):
            for k in range(D // 16):
                sl = pl.ds(k * 16, 16)
                plsc.addupdate(rows_v.at[j, r, sl], pos_v[r, sl])

    def chunk_step(i, j, with_wait_out=True, with_start_gather=True):
        if with_wait_out:
            wait_out((j + 2) % NBUF)
        if with_start_gather:
            start_gather(i + 2, (j + 2) % NBUF)
        wait_gather(j)
        add_pos(j)
        start_out(i, j)

    start_gather(0, 0)
    start_gather(1, 1)

    chunk_step(0, 0, with_wait_out=False)
    chunk_step(1, 1, with_wait_out=False)
    chunk_step(2, 2)
    chunk_step(3, 3)

    @pl.loop(NBUF, B_PER_W - NBUF, step=NBUF)
    def _ring(i0):
        for j in range(NBUF):
            chunk_step(i0 + j, j)

    i_last = B_PER_W - NBUF
    chunk_step(i_last + 0, 0)
    chunk_step(i_last + 1, 1)
    chunk_step(i_last + 2, 2, with_start_gather=False)
    chunk_step(i_last + 3, 3, with_start_gather=False)

    wait_out((i_last + 2) % NBUF)
    wait_out((i_last + 3) % NBUF)


def _emb(idx2, token_table, pos_table):
    mesh = plsc.VectorSubcoreMesh(
        core_axis_name="c", subcore_axis_name="s",
        num_cores=NUM_CORES, num_subcores=NUM_SUBCORES)
    return pl.kernel(
        _emb_body,
        out_type=jax.ShapeDtypeStruct((B, L, D), jnp.float32),
        mesh=mesh,
        scratch_types=[
            pltpu.VMEM((L, D), jnp.float32),
            pltpu.VMEM((2 * B_PER_W, HALF), jnp.int32),
            pltpu.VMEM((NBUF, L, D), jnp.float32),
            pltpu.SemaphoreType.DMA((NBUF,)),
            pltpu.SemaphoreType.DMA((NBUF,)),
        ],
        compiler_params=pltpu.CompilerParams(use_tc_tiling_on_sc=False),
    )(idx2, token_table, pos_table)


def kernel(input, token_table, pos_table):
    idx2 = input.reshape(B * L // HALF, HALF)
    return _emb(idx2, token_table, pos_table)
